# baseline (speedup 1.0000x reference)
import functools

import jax
import jax.numpy as jnp
from jax import lax
from jax.experimental import pallas as pl
from jax.experimental.pallas import tpu as pltpu
from jax.experimental.pallas import tpu_sc as plsc

_NUM_CORES = 2
_NUM_SUBCORES = 16
_NW = _NUM_CORES * _NUM_SUBCORES


@jax.jit
def _embed(xw, table):
    nw, hist, b_per_w = xw.shape
    vocab, d = table.shape
    batch = nw * b_per_w

    mesh = plsc.VectorSubcoreMesh(
        core_axis_name="c",
        subcore_axis_name="s",
        num_cores=_NUM_CORES,
        num_subcores=_NUM_SUBCORES,
    )

    nbuf = 2
    lookahead = 1

    @functools.partial(
        pl.kernel,
        out_type=jax.ShapeDtypeStruct((hist * batch, d), jnp.float32),
        mesh=mesh,
        compiler_params=pltpu.CompilerParams(use_tc_tiling_on_sc=True),
        scratch_types=[
            pltpu.VMEM((hist, b_per_w), jnp.int32),
            [pltpu.VMEM((b_per_w, d), jnp.float32) for _ in range(nbuf)],
            [pltpu.SemaphoreType.DMA for _ in range(nbuf)],
            [pltpu.SemaphoreType.DMA for _ in range(nbuf)],
        ],
    )
    def embed_kernel(x_hbm, table_hbm, out_hbm, idx_v, bufs, sems_g, sems_s):
        wid = lax.axis_index("s") * _NUM_CORES + lax.axis_index("c")
        base = wid * b_per_w

        pltpu.sync_copy(x_hbm.at[wid], idx_v)

        for b in range(lookahead):
            pltpu.async_copy(table_hbm.at[idx_v.at[b]], bufs[b], sems_g[b])

        def outer(g, carry):
            for b in range(nbuf):
                h = g * nbuf + b
                pltpu.make_async_copy(
                    table_hbm.at[idx_v.at[h]], bufs[b], sems_g[b]
                ).wait()
                pltpu.async_copy(
                    bufs[b], out_hbm.at[pl.ds(h * batch + base, b_per_w)], sems_s[b]
                )

                bf = (b + lookahead) % nbuf

                @pl.when(h + lookahead < hist)
                def _prefetch():
                    @pl.when(h + lookahead >= nbuf)
                    def _drain_prev_scatter():
                        pltpu.make_async_copy(
                            bufs[bf],
                            out_hbm.at[pl.ds(base, b_per_w)],
                            sems_s[bf],
                        ).wait()

                    pltpu.async_copy(
                        table_hbm.at[idx_v.at[h + lookahead]], bufs[bf], sems_g[bf]
                    )

            return carry

        lax.fori_loop(0, hist // nbuf, outer, None)

        for b in range(nbuf):
            pltpu.make_async_copy(
                bufs[b], out_hbm.at[pl.ds(base, b_per_w)], sems_s[b]
            ).wait()

    return embed_kernel(xw, table)


def kernel(x, table):
    batch, hist = x.shape
    xw = x.astype(jnp.int32).T.reshape(hist, _NW, batch // _NW).transpose(1, 0, 2)
    out = _embed(xw, table)
    return out.reshape(hist, batch, table.shape[1]).swapaxes(0, 1)


# speedup vs baseline: 1.2230x; 1.2230x over previous
import functools

import jax
import jax.numpy as jnp
from jax import lax
from jax.experimental import pallas as pl
from jax.experimental.pallas import tpu as pltpu
from jax.experimental.pallas import tpu_sc as plsc

_NUM_CORES = 2
_NUM_SUBCORES = 16
_NW = _NUM_CORES * _NUM_SUBCORES


@functools.partial(jax.jit, static_argnums=2)
def _embed(xw, table, hist):
    nw, n_chunks, chunk = xw.shape
    vocab, d = table.shape
    per_h = n_chunks // hist
    b_per_w = per_h * chunk
    batch = nw * b_per_w

    mesh = plsc.VectorSubcoreMesh(
        core_axis_name="c",
        subcore_axis_name="s",
        num_cores=_NUM_CORES,
        num_subcores=_NUM_SUBCORES,
    )

    nbuf = 10
    lookahead = 6

    @functools.partial(
        pl.kernel,
        out_type=jax.ShapeDtypeStruct((hist * batch, d), jnp.float32),
        mesh=mesh,
        compiler_params=pltpu.CompilerParams(use_tc_tiling_on_sc=True),
        scratch_types=[
            pltpu.VMEM((n_chunks, chunk), jnp.int32),
            [pltpu.VMEM((chunk, d), jnp.float32) for _ in range(nbuf)],
            [pltpu.SemaphoreType.DMA for _ in range(nbuf)],
            [pltpu.SemaphoreType.DMA for _ in range(nbuf)],
        ],
    )
    def embed_kernel(x_hbm, table_hbm, out_hbm, idx_v, bufs, sems_g, sems_s):
        wid = lax.axis_index("s") * _NUM_CORES + lax.axis_index("c")
        base = wid * b_per_w

        pltpu.sync_copy(x_hbm.at[wid], idx_v)

        def out_off(c):
            return (c // per_h) * batch + base + (c % per_h) * chunk

        for b in range(lookahead):
            pltpu.async_copy(table_hbm.at[idx_v.at[b]], bufs[b], sems_g[b])

        def outer(g, carry):
            for b in range(nbuf):
                c = g * nbuf + b
                pltpu.make_async_copy(
                    table_hbm.at[idx_v.at[c]], bufs[b], sems_g[b]
                ).wait()
                pltpu.async_copy(
                    bufs[b], out_hbm.at[pl.ds(out_off(c), chunk)], sems_s[b]
                )

                bf = (b + lookahead) % nbuf

                @pl.when(c + lookahead < n_chunks)
                def _prefetch():
                    @pl.when(c + lookahead >= nbuf)
                    def _drain_prev_scatter():
                        pltpu.make_async_copy(
                            bufs[bf],
                            out_hbm.at[pl.ds(base, chunk)],
                            sems_s[bf],
                        ).wait()

                    pltpu.async_copy(
                        table_hbm.at[idx_v.at[c + lookahead]], bufs[bf], sems_g[bf]
                    )

            return carry

        lax.fori_loop(0, n_chunks // nbuf, outer, None)

        for b in range(nbuf):
            pltpu.make_async_copy(
                bufs[b], out_hbm.at[pl.ds(base, chunk)], sems_s[b]
            ).wait()

    return embed_kernel(xw, table)


def kernel(x, table):
    batch, hist = x.shape
    chunk = 64
    per_h = batch // _NW // chunk
    xw = (
        x.astype(jnp.int32)
        .T.reshape(hist, _NW, per_h, chunk)
        .transpose(1, 0, 2, 3)
        .reshape(_NW, hist * per_h, chunk)
    )
    out = _embed(xw, table, hist)
    return out.reshape(hist, batch, table.shape[1]).swapaxes(0, 1)


# speedup vs baseline: 1.2524x; 1.0240x over previous
import functools

import jax
import jax.numpy as jnp
from jax import lax
from jax.experimental import pallas as pl
from jax.experimental.pallas import tpu as pltpu
from jax.experimental.pallas import tpu_sc as plsc

_NUM_CORES = 2
_NUM_SUBCORES = 16
_NW = _NUM_CORES * _NUM_SUBCORES


@jax.jit
def _embed(xw, table):
    nw, hist, b_per_w = xw.shape
    vocab, d = table.shape
    batch = nw * b_per_w

    mesh = plsc.VectorSubcoreMesh(
        core_axis_name="c",
        subcore_axis_name="s",
        num_cores=_NUM_CORES,
        num_subcores=_NUM_SUBCORES,
    )

    nbuf = 5
    lookahead = 3

    @functools.partial(
        pl.kernel,
        out_type=jax.ShapeDtypeStruct((hist * batch, d), jnp.float32),
        mesh=mesh,
        compiler_params=pltpu.CompilerParams(use_tc_tiling_on_sc=True),
        scratch_types=[
            pltpu.VMEM((hist, b_per_w), jnp.int32),
            [pltpu.VMEM((b_per_w, d), jnp.float32) for _ in range(nbuf)],
            [pltpu.SemaphoreType.DMA for _ in range(nbuf)],
            [pltpu.SemaphoreType.DMA for _ in range(nbuf)],
        ],
    )
    def embed_kernel(x_hbm, table_hbm, out_hbm, idx_v, bufs, sems_g, sems_s):
        wid = lax.axis_index("s") * _NUM_CORES + lax.axis_index("c")
        base = wid * b_per_w

        pltpu.sync_copy(x_hbm.at[wid], idx_v)

        for b in range(lookahead):
            pltpu.async_copy(table_hbm.at[idx_v.at[b]], bufs[b], sems_g[b])

        def outer(g, carry):
            for b in range(nbuf):
                h = g * nbuf + b
                pltpu.make_async_copy(
                    table_hbm.at[idx_v.at[h]], bufs[b], sems_g[b]
                ).wait()
                pltpu.async_copy(
                    bufs[b], out_hbm.at[pl.ds(h * batch + base, b_per_w)], sems_s[b]
                )

                bf = (b + lookahead) % nbuf

                @pl.when(h + lookahead < hist)
                def _prefetch():
                    @pl.when(h + lookahead >= nbuf)
                    def _drain_prev_scatter():
                        pltpu.make_async_copy(
                            bufs[bf],
                            out_hbm.at[pl.ds(base, b_per_w)],
                            sems_s[bf],
                        ).wait()

                    pltpu.async_copy(
                        table_hbm.at[idx_v.at[h + lookahead]], bufs[bf], sems_g[bf]
                    )

            return carry

        lax.fori_loop(0, hist // nbuf, outer, None)

        for b in range(nbuf):
            pltpu.make_async_copy(
                bufs[b], out_hbm.at[pl.ds(base, b_per_w)], sems_s[b]
            ).wait()

    return embed_kernel(xw, table)


def kernel(x, table):
    batch, hist = x.shape
    xw = x.astype(jnp.int32).T.reshape(hist, _NW, batch // _NW).transpose(1, 0, 2)
    out = _embed(xw, table)
    return out.reshape(hist, batch, table.shape[1]).swapaxes(0, 1)


# speedup vs baseline: 1.2540x; 1.0012x over previous
import functools

import jax
import jax.numpy as jnp
from jax import lax
from jax.experimental import pallas as pl
from jax.experimental.pallas import tpu as pltpu
from jax.experimental.pallas import tpu_sc as plsc

_NUM_CORES = 2
_NUM_SUBCORES = 16
_NW = _NUM_CORES * _NUM_SUBCORES


@jax.jit
def _embed(xw, table):
    nw, hist, b_per_w = xw.shape
    vocab, d = table.shape
    batch = nw * b_per_w

    mesh = plsc.VectorSubcoreMesh(
        core_axis_name="c",
        subcore_axis_name="s",
        num_cores=_NUM_CORES,
        num_subcores=_NUM_SUBCORES,
    )

    nbuf = 5
    lookahead = 3

    @functools.partial(
        pl.kernel,
        out_type=jax.ShapeDtypeStruct((hist * batch, d), jnp.float32),
        mesh=mesh,
        compiler_params=pltpu.CompilerParams(
            use_tc_tiling_on_sc=True,
            disable_bounds_checks=True,
            disable_semaphore_checks=True,
        ),
        scratch_types=[
            pltpu.VMEM((hist, b_per_w), jnp.int32),
            [pltpu.VMEM((b_per_w, d), jnp.float32) for _ in range(nbuf)],
            [pltpu.SemaphoreType.DMA for _ in range(nbuf)],
            [pltpu.SemaphoreType.DMA for _ in range(nbuf)],
        ],
    )
    def embed_kernel(x_hbm, table_hbm, out_hbm, idx_v, bufs, sems_g, sems_s):
        wid = lax.axis_index("s") * _NUM_CORES + lax.axis_index("c")
        base = wid * b_per_w

        pltpu.sync_copy(x_hbm.at[wid], idx_v)

        for b in range(lookahead):
            pltpu.async_copy(table_hbm.at[idx_v.at[b]], bufs[b], sems_g[b])

        def outer(g, carry):
            for b in range(nbuf):
                h = g * nbuf + b
                pltpu.make_async_copy(
                    table_hbm.at[idx_v.at[h]], bufs[b], sems_g[b]
                ).wait()
                pltpu.async_copy(
                    bufs[b], out_hbm.at[pl.ds(h * batch + base, b_per_w)], sems_s[b]
                )

                bf = (b + lookahead) % nbuf

                @pl.when(h + lookahead < hist)
                def _prefetch():
                    @pl.when(h + lookahead >= nbuf)
                    def _drain_prev_scatter():
                        pltpu.make_async_copy(
                            bufs[bf],
                            out_hbm.at[pl.ds(base, b_per_w)],
                            sems_s[bf],
                        ).wait()

                    pltpu.async_copy(
                        table_hbm.at[idx_v.at[h + lookahead]], bufs[bf], sems_g[bf]
                    )

            return carry

        lax.fori_loop(0, hist // nbuf, outer, None)

        for b in range(nbuf):
            pltpu.make_async_copy(
                bufs[b], out_hbm.at[pl.ds(base, b_per_w)], sems_s[b]
            ).wait()

    return embed_kernel(xw, table)


def kernel(x, table):
    batch, hist = x.shape
    xw = x.astype(jnp.int32).T.reshape(hist, _NW, batch // _NW).transpose(1, 0, 2)
    out = _embed(xw, table)
    return out.reshape(hist, batch, table.shape[1]).swapaxes(0, 1)
